# Initial kernel scaffold; baseline (speedup 1.0000x reference)
import jax
import jax.numpy as jnp
from jax.experimental import pallas as pl


def kernel(x, table):
    raise NotImplementedError("write your pallas kernel here")


# speedup vs baseline: 2.9695x; 2.9695x over previous
import functools

import jax
import jax.numpy as jnp
from jax import lax
from jax.experimental import pallas as pl
from jax.experimental.pallas import tpu as pltpu
from jax.experimental.pallas import tpu_sc as plsc

EMBED = 128
IDX_ROW = 128


def _make_gather(num_idx: int, vocab: int):
  info = plsc.get_sparse_core_info()
  nc, ns = info.num_cores, info.num_subcores
  nw = nc * ns
  assert num_idx % (nw * IDX_ROW) == 0
  rows_per_w = num_idx // (nw * IDX_ROW)

  mesh = plsc.VectorSubcoreMesh(core_axis_name="c", subcore_axis_name="s")

  @functools.partial(
      pl.kernel,
      mesh=mesh,
      out_type=jax.ShapeDtypeStruct((num_idx, EMBED), jnp.float32),
      scratch_types=[
          pltpu.VMEM((rows_per_w, IDX_ROW), jnp.int32),
          pltpu.VMEM((IDX_ROW, EMBED), jnp.float32),
          pltpu.SemaphoreType.DMA,
      ],
  )
  def gather_kernel(idx_hbm, table_hbm, out_hbm, idx_v, rows_v, sem):
    wid = lax.axis_index("s") * nc + lax.axis_index("c")
    pltpu.sync_copy(idx_hbm.at[wid], idx_v)
    base = wid * rows_per_w * IDX_ROW

    def body(j, carry):
      off = pl.multiple_of(base + j * IDX_ROW, IDX_ROW)
      pltpu.async_copy(table_hbm.at[idx_v.at[j]], rows_v, sem).wait()
      pltpu.sync_copy(rows_v, out_hbm.at[pl.ds(off, IDX_ROW)])
      return carry

    lax.fori_loop(0, rows_per_w, body, 0)

  return gather_kernel


def kernel(x, table):
  num_idx = x.size
  idx = x.reshape(-1).astype(jnp.int32)
  info = plsc.get_sparse_core_info()
  nw = info.num_cores * info.num_subcores
  rows_per_w = num_idx // (nw * IDX_ROW)
  idx3 = idx.reshape(nw, rows_per_w, IDX_ROW)
  out = _make_gather(num_idx, table.shape[0])(idx3, table)
  return out.reshape(x.shape + (EMBED,))


# speedup vs baseline: 3.3480x; 1.1275x over previous
import functools

import jax
import jax.numpy as jnp
from jax import lax
from jax.experimental import pallas as pl
from jax.experimental.pallas import tpu as pltpu
from jax.experimental.pallas import tpu_sc as plsc

EMBED = 128
IDX_ROW = 128
NBUF = 5
AHEAD = NBUF - 2


def _make_gather(num_idx: int):
  info = plsc.get_sparse_core_info()
  nc, ns = info.num_cores, info.num_subcores
  nw = nc * ns
  assert num_idx % (nw * IDX_ROW) == 0
  rows_per_w = num_idx // (nw * IDX_ROW)
  assert rows_per_w % NBUF == 0 or rows_per_w > 2 * NBUF

  mesh = plsc.VectorSubcoreMesh(core_axis_name="c", subcore_axis_name="s")

  @functools.partial(
      pl.kernel,
      mesh=mesh,
      out_type=jax.ShapeDtypeStruct((num_idx, EMBED), jnp.float32),
      scratch_types=(
          [pltpu.VMEM((rows_per_w, IDX_ROW), jnp.int32)]
          + [pltpu.VMEM((IDX_ROW, EMBED), jnp.float32) for _ in range(NBUF)]
          + [pltpu.SemaphoreType.DMA for _ in range(2 * NBUF)]
      ),
  )
  def gather_kernel(idx_hbm, table_hbm, out_hbm, idx_v, *rest):
    bufs = rest[:NBUF]
    gsem = rest[NBUF:2 * NBUF]
    ssem = rest[2 * NBUF:]
    wid = lax.axis_index("s") * nc + lax.axis_index("c")
    pltpu.sync_copy(idx_hbm.at[wid], idx_v)
    base = wid * rows_per_w * IDX_ROW

    def g_start(b, j):
      pltpu.async_copy(table_hbm.at[idx_v.at[j]], bufs[b], gsem[b])

    def g_wait(b):
      pltpu.make_async_copy(
          table_hbm.at[pl.ds(0, IDX_ROW)], bufs[b], gsem[b]).wait()

    def s_start(b, j):
      off = pl.multiple_of(base + j * IDX_ROW, IDX_ROW)
      pltpu.async_copy(bufs[b], out_hbm.at[pl.ds(off, IDX_ROW)], ssem[b])

    def s_wait(b):
      pltpu.make_async_copy(
          bufs[b], out_hbm.at[pl.ds(base, IDX_ROW)], ssem[b]).wait()

    T = rows_per_w
    for j in range(AHEAD):
      g_start(j % NBUF, j)
    for j in range(2):
      g_start((j + AHEAD) % NBUF, j + AHEAD)
      g_wait(j % NBUF)
      s_start(j % NBUF, j)

    n_main = T - 2 - (AHEAD + 2 - 2)
    main_lo, main_hi = 2, T - AHEAD
    n_iters = main_hi - main_lo
    n_outer = n_iters // NBUF
    n_rem = n_iters % NBUF

    def outer(t, carry):
      for i in range(NBUF):
        j = main_lo + t * NBUF + i
        b = (main_lo + i) % NBUF
        s_wait((b - 2) % NBUF)
        g_start((b + AHEAD) % NBUF, j + AHEAD)
        g_wait(b)
        s_start(b, j)
      return carry

    lax.fori_loop(0, n_outer, outer, 0)
    for k in range(n_rem):
      j = main_lo + n_outer * NBUF + k
      b = j % NBUF
      s_wait((b - 2) % NBUF)
      g_start((b + AHEAD) % NBUF, j + AHEAD)
      g_wait(b)
      s_start(b, j)
    for j in range(T - AHEAD, T):
      b = j % NBUF
      s_wait((b - 2) % NBUF)
      g_wait(b)
      s_start(b, j)
    s_wait((T - 2) % NBUF)
    s_wait((T - 1) % NBUF)

  return gather_kernel


def kernel(x, table):
  num_idx = x.size
  idx = x.reshape(-1).astype(jnp.int32)
  info = plsc.get_sparse_core_info()
  nw = info.num_cores * info.num_subcores
  rows_per_w = num_idx // (nw * IDX_ROW)
  idx3 = idx.reshape(nw, rows_per_w, IDX_ROW)
  out = _make_gather(num_idx)(idx3, table)
  return out.reshape(x.shape + (EMBED,))


# speedup vs baseline: 5.9672x; 1.7823x over previous
import functools

import jax
import jax.numpy as jnp
from jax import lax
from jax.experimental import pallas as pl
from jax.experimental.pallas import tpu as pltpu
from jax.experimental.pallas import tpu_sc as plsc

EMBED = 128
SEQ = 50
SC_K = 4
NBUF = 4
AHEAD = NBUF - 2


def _make_gather(num_rows: int):
  info = plsc.get_sparse_core_info()
  nc, ns = info.num_cores, info.num_subcores
  nw = nc * ns
  assert num_rows % (nw * SC_K) == 0
  rows_per_w = num_rows // nw
  T = rows_per_w // SC_K

  mesh = plsc.VectorSubcoreMesh(core_axis_name="c", subcore_axis_name="s")

  @functools.partial(
      pl.kernel,
      mesh=mesh,
      out_type=jax.ShapeDtypeStruct((num_rows, SEQ, EMBED), jnp.float32),
      scratch_types=(
          [pltpu.VMEM((rows_per_w, SEQ), jnp.int32)]
          + [pltpu.VMEM((SC_K, SEQ, EMBED), jnp.float32) for _ in range(NBUF)]
          + [pltpu.SemaphoreType.DMA for _ in range(2 * NBUF)]
      ),
  )
  def gather_kernel(idx_hbm, table_hbm, out_hbm, idx_v, *rest):
    bufs = rest[:NBUF]
    gsem = rest[NBUF:2 * NBUF]
    ssem = rest[2 * NBUF:]
    wid = lax.axis_index("s") * nc + lax.axis_index("c")
    pltpu.sync_copy(idx_hbm.at[wid], idx_v)
    row0 = wid * rows_per_w

    def g_start(b, j):
      for i in range(SC_K):
        pltpu.async_copy(
            table_hbm.at[idx_v.at[j * SC_K + i]], bufs[b].at[i], gsem[b])

    def g_wait(b):
      pltpu.make_async_copy(
          out_hbm.at[pl.ds(0, SC_K)], bufs[b], gsem[b]).wait()

    def s_start(b, j):
      pltpu.async_copy(
          bufs[b], out_hbm.at[pl.ds(row0 + j * SC_K, SC_K)], ssem[b])

    def s_wait(b):
      pltpu.make_async_copy(
          bufs[b], out_hbm.at[pl.ds(0, SC_K)], ssem[b]).wait()

    for j in range(AHEAD):
      g_start(j % NBUF, j)
    for j in range(2):
      g_start((j + AHEAD) % NBUF, j + AHEAD)
      g_wait(j % NBUF)
      s_start(j % NBUF, j)

    main_lo, main_hi = 2, T - AHEAD
    n_iters = main_hi - main_lo
    n_outer = n_iters // NBUF
    n_rem = n_iters % NBUF

    def outer(t, carry):
      for i in range(NBUF):
        j = main_lo + t * NBUF + i
        b = (main_lo + i) % NBUF
        s_wait((b - 2) % NBUF)
        g_start((b + AHEAD) % NBUF, j + AHEAD)
        g_wait(b)
        s_start(b, j)
      return carry

    lax.fori_loop(0, n_outer, outer, 0)
    for k in range(n_rem):
      j = main_lo + n_outer * NBUF + k
      b = (main_lo + k) % NBUF
      s_wait((b - 2) % NBUF)
      g_start((b + AHEAD) % NBUF, j + AHEAD)
      g_wait(b)
      s_start(b, j)
    for j in range(T - AHEAD, T):
      b = j % NBUF
      s_wait((b - 2) % NBUF)
      g_wait(b)
      s_start(b, j)
    s_wait((T - 2) % NBUF)
    s_wait((T - 1) % NBUF)

  return gather_kernel


def kernel(x, table):
  num_rows, seq = x.shape
  assert seq == SEQ
  info = plsc.get_sparse_core_info()
  nw = info.num_cores * info.num_subcores
  idx3 = x.astype(jnp.int32).reshape(nw, num_rows // nw, SEQ)
  return _make_gather(num_rows)(idx3, table)


# speedup vs baseline: 5.9883x; 1.0035x over previous
import functools

import jax
import jax.numpy as jnp
from jax import lax
from jax.experimental import pallas as pl
from jax.experimental.pallas import tpu as pltpu
from jax.experimental.pallas import tpu_sc as plsc

EMBED = 128
SEQ = 50
SC_K = 4
NBUF = 4
AHEAD = NBUF - 2


def _make_gather(num_rows: int):
  info = plsc.get_sparse_core_info()
  nc, ns = info.num_cores, info.num_subcores
  nw = nc * ns
  assert num_rows % (nw * SC_K) == 0
  rows_per_w = num_rows // nw
  T = rows_per_w // SC_K

  mesh = plsc.VectorSubcoreMesh(core_axis_name="c", subcore_axis_name="s")

  @functools.partial(
      pl.kernel,
      mesh=mesh,
      compiler_params=pltpu.CompilerParams(use_tc_tiling_on_sc=True),
      out_type=jax.ShapeDtypeStruct((num_rows, SEQ, EMBED), jnp.float32),
      scratch_types=(
          [pltpu.VMEM((rows_per_w, SEQ), jnp.int32)]
          + [pltpu.VMEM((SC_K, SEQ, EMBED), jnp.float32) for _ in range(NBUF)]
          + [pltpu.SemaphoreType.DMA for _ in range(2 * NBUF)]
      ),
  )
  def gather_kernel(idx_hbm, table_hbm, out_hbm, idx_v, *rest):
    bufs = rest[:NBUF]
    gsem = rest[NBUF:2 * NBUF]
    ssem = rest[2 * NBUF:]
    wid = lax.axis_index("s") * nc + lax.axis_index("c")
    pltpu.sync_copy(idx_hbm.at[wid], idx_v)
    row0 = wid * rows_per_w

    def g_start(b, j):
      for i in range(SC_K):
        pltpu.async_copy(
            table_hbm.at[idx_v.at[j * SC_K + i]], bufs[b].at[i], gsem[b])

    def g_wait(b):
      pltpu.make_async_copy(
          out_hbm.at[pl.ds(0, SC_K)], bufs[b], gsem[b]).wait()

    def s_start(b, j):
      pltpu.async_copy(
          bufs[b], out_hbm.at[pl.ds(row0 + j * SC_K, SC_K)], ssem[b])

    def s_wait(b):
      pltpu.make_async_copy(
          bufs[b], out_hbm.at[pl.ds(0, SC_K)], ssem[b]).wait()

    for j in range(AHEAD):
      g_start(j % NBUF, j)
    for j in range(2):
      g_start((j + AHEAD) % NBUF, j + AHEAD)
      g_wait(j % NBUF)
      s_start(j % NBUF, j)

    main_lo, main_hi = 2, T - AHEAD
    n_iters = main_hi - main_lo
    n_outer = n_iters // NBUF
    n_rem = n_iters % NBUF

    def outer(t, carry):
      for i in range(NBUF):
        j = main_lo + t * NBUF + i
        b = (main_lo + i) % NBUF
        s_wait((b - 2) % NBUF)
        g_start((b + AHEAD) % NBUF, j + AHEAD)
        g_wait(b)
        s_start(b, j)
      return carry

    lax.fori_loop(0, n_outer, outer, 0)
    for k in range(n_rem):
      j = main_lo + n_outer * NBUF + k
      b = (main_lo + k) % NBUF
      s_wait((b - 2) % NBUF)
      g_start((b + AHEAD) % NBUF, j + AHEAD)
      g_wait(b)
      s_start(b, j)
    for j in range(T - AHEAD, T):
      b = j % NBUF
      s_wait((b - 2) % NBUF)
      g_wait(b)
      s_start(b, j)
    s_wait((T - 2) % NBUF)
    s_wait((T - 1) % NBUF)

  return gather_kernel


def kernel(x, table):
  num_rows, seq = x.shape
  assert seq == SEQ
  info = plsc.get_sparse_core_info()
  nw = info.num_cores * info.num_subcores
  idx3 = x.astype(jnp.int32).reshape(nw, num_rows // nw, SEQ)
  return _make_gather(num_rows)(idx3, table)
